# baseline (speedup 1.0000x reference)
import jax
import jax.numpy as jnp
from jax.experimental import pallas as pl

G0 = 256
G1 = 256
D = 256
ROWS_PER_BLOCK = 8


def _fill_body(vals_ref, out_ref):
    i = pl.program_id(0)
    rows = jax.lax.broadcasted_iota(jnp.int32, (ROWS_PER_BLOCK, G1, 1), 0)
    cols = jax.lax.broadcasted_iota(jnp.int32, (ROWS_PER_BLOCK, G1, 1), 1)
    pred = (rows + i * ROWS_PER_BLOCK + cols) >= (G0 - 1)
    lo = vals_ref[0, :][None, None, :]
    hi = vals_ref[1, :][None, None, :]
    out_ref[...] = jnp.where(pred, hi, lo)


def kernel(left_to_right, right_to_left, top_to_bottom, bottom_to_top,
           top_left_to_bottom_right, bottom_right_to_top_left,
           bottom_left_to_top_right, top_right_to_bottom_left):
    vals = jnp.stack([bottom_left_to_top_right, top_right_to_bottom_left])
    return pl.pallas_call(
        _fill_body,
        grid=(G0 // ROWS_PER_BLOCK,),
        in_specs=[pl.BlockSpec((2, D), lambda i: (0, 0))],
        out_specs=pl.BlockSpec((ROWS_PER_BLOCK, G1, D), lambda i: (i, 0, 0)),
        out_shape=jax.ShapeDtypeStruct((G0, G1, D), jnp.float32),
    )(vals)


# speedup vs baseline: 1.1050x; 1.1050x over previous
import jax
import jax.numpy as jnp
from jax.experimental import pallas as pl
from jax.experimental.pallas import tpu as pltpu

G0 = 256
G1 = 256
D = 256
T_ROWS = 504


def _dma_body(vals_ref, out_ref, tabs_ref, sem):
    for k in range(8):
        rows = jax.lax.broadcasted_iota(jnp.int32, (T_ROWS, 1), 0) + k
        tabs_ref[k] = jnp.where(rows < (G0 - 1),
                                vals_ref[0, :][None, :],
                                vals_ref[1, :][None, :])
    copies = []
    for r in range(G0):
        k = r % 8
        q8 = r - k
        copies.append(
            pltpu.make_async_copy(tabs_ref.at[k, pl.ds(q8, G1)],
                                  out_ref.at[r], sem))
    for c in copies:
        c.start()
    for c in copies:
        c.wait()


def kernel(left_to_right, right_to_left, top_to_bottom, bottom_to_top,
           top_left_to_bottom_right, bottom_right_to_top_left,
           bottom_left_to_top_right, top_right_to_bottom_left):
    vals = jnp.stack([bottom_left_to_top_right, top_right_to_bottom_left])
    return pl.pallas_call(
        _dma_body,
        in_specs=[pl.BlockSpec((2, D), lambda: (0, 0))],
        out_specs=pl.BlockSpec(memory_space=pl.ANY),
        out_shape=jax.ShapeDtypeStruct((G0, G1, D), jnp.float32),
        scratch_shapes=[
            pltpu.VMEM((8, T_ROWS, D), jnp.float32),
            pltpu.SemaphoreType.DMA,
        ],
    )(vals)


# speedup vs baseline: 1.1700x; 1.0589x over previous
import jax
import jax.numpy as jnp
from jax.experimental import pallas as pl

G0 = 256
G1 = 256
D = 256
ROWS_PER_BLOCK = 16


def _fill_body(vals_ref, out_ref):
    i = pl.program_id(0)
    rows = jax.lax.broadcasted_iota(jnp.int32, (ROWS_PER_BLOCK, G1, 1), 0)
    cols = jax.lax.broadcasted_iota(jnp.int32, (ROWS_PER_BLOCK, G1, 1), 1)
    pred = (rows + i * ROWS_PER_BLOCK + cols) >= (G0 - 1)
    lo = vals_ref[0, :][None, None, :]
    hi = vals_ref[1, :][None, None, :]
    out_ref[...] = jnp.where(pred, hi, lo)


def kernel(left_to_right, right_to_left, top_to_bottom, bottom_to_top,
           top_left_to_bottom_right, bottom_right_to_top_left,
           bottom_left_to_top_right, top_right_to_bottom_left):
    vals = jnp.stack([bottom_left_to_top_right, top_right_to_bottom_left])
    return pl.pallas_call(
        _fill_body,
        grid=(G0 // ROWS_PER_BLOCK,),
        in_specs=[pl.BlockSpec((2, D), lambda i: (0, 0))],
        out_specs=pl.BlockSpec((ROWS_PER_BLOCK, G1, D), lambda i: (i, 0, 0)),
        out_shape=jax.ShapeDtypeStruct((G0, G1, D), jnp.float32),
    )(vals)
